# baseline (speedup 1.0000x reference)
import functools

import jax
import jax.numpy as jnp
from jax import lax
from jax.experimental import pallas as pl
from jax.experimental.pallas import tpu as pltpu
from jax.experimental.pallas import tpu_sc as plsc

NUM_SPARSE = 26
VOCAB_P1 = 100001
EMB_DIM = 16
BATCH = 4096
F = NUM_SPARSE + 1

NW = 32
CHUNK = 128
ROWS = NUM_SPARSE * BATCH
ROWS_PW = ROWS // NW
CHUNKS_PW = ROWS_PW // CHUNK


def _sc_gather(tables_flat, ids_flat):
    mesh = plsc.VectorSubcoreMesh(core_axis_name="c", subcore_axis_name="s")

    @functools.partial(
        pl.kernel,
        out_type=jax.ShapeDtypeStruct((ROWS, EMB_DIM), jnp.float32),
        mesh=mesh,
        scratch_types=[
            pltpu.VMEM((CHUNK,), jnp.int32),
            pltpu.VMEM((CHUNK, EMB_DIM), jnp.float32),
            pltpu.SemaphoreType.DMA,
        ],
        compiler_params=pltpu.CompilerParams(use_tc_tiling_on_sc=False),
        name="dlrm_sc_gather",
    )
    def gather_kernel(tab_hbm, ids_hbm, out_hbm, idx_v, rows_v, sem):
        wid = lax.axis_index("s") * 2 + lax.axis_index("c")
        base = wid * ROWS_PW

        def body(i, carry):
            off = base + i * CHUNK
            pltpu.sync_copy(ids_hbm.at[pl.ds(off, CHUNK)], idx_v)
            pltpu.async_copy(tab_hbm.at[idx_v], rows_v, sem).wait()
            pltpu.sync_copy(rows_v, out_hbm.at[pl.ds(off, CHUNK)])
            return carry

        lax.fori_loop(0, CHUNKS_PW, body, 0)

    return gather_kernel(tables_flat, ids_flat)


BB = 512


def _tc_body(dense_ref, g_ref, mask_ref, bW1_ref, bb1_ref, bW2_ref, bb2_ref,
             bW3_ref, bb3_ref, tW1a_ref, tW1b_ref, tb1_ref, tW2_ref, tb2_ref,
             tW3_ref, tb3_ref, out_ref):
    f32 = jnp.float32
    d = jnp.dot(dense_ref[...], bW1_ref[...], preferred_element_type=f32) + bb1_ref[...]
    d = jnp.dot(d, bW2_ref[...], preferred_element_type=f32) + bb2_ref[...]
    d = jnp.maximum(jnp.dot(d, bW3_ref[...], preferred_element_type=f32) + bb3_ref[...], 0.0)
    feats = jnp.concatenate([g_ref[...], d[None]], axis=0)
    xact = lax.dot_general(feats, feats, (((2,), (2,)), ((1,), (1,))))
    xm = xact * mask_ref[...][None]
    t = jnp.dot(d, tW1a_ref[...], preferred_element_type=f32)
    t = t + jnp.dot(xm.reshape(xm.shape[0], F * F), tW1b_ref[...],
                    preferred_element_type=f32)
    t = t + tb1_ref[...]
    t = jnp.dot(t, tW2_ref[...], preferred_element_type=f32) + tb2_ref[...]
    t = jnp.dot(t, tW3_ref[...], preferred_element_type=f32) + tb3_ref[...]
    out_ref[...] = jax.nn.sigmoid(t)


def _whole(shape):
    return pl.BlockSpec(shape, lambda i: (0,) * len(shape))


def kernel(dense_features, sparse_ids, emb_tables, bW1, bb1, bW2, bb2, bW3,
           bb3, tW1, tb1, tW2, tb2, tW3, tb3):
    tables_flat = emb_tables.reshape(NUM_SPARSE * VOCAB_P1, EMB_DIM)
    offs = (jnp.arange(NUM_SPARSE, dtype=jnp.int32) * VOCAB_P1)[:, None]
    ids_flat = (sparse_ids.astype(jnp.int32) + offs).reshape(ROWS)
    gathered = _sc_gather(tables_flat, ids_flat).reshape(NUM_SPARSE, BATCH, EMB_DIM)

    mask = (lax.broadcasted_iota(jnp.int32, (F, F), 0)
            > lax.broadcasted_iota(jnp.int32, (F, F), 1)).astype(jnp.float32)
    tW1a = tW1[:EMB_DIM]
    tW1b = tW1[EMB_DIM:]

    nb = BATCH // BB
    out = pl.pallas_call(
        _tc_body,
        grid=(nb,),
        in_specs=[
            pl.BlockSpec((BB, 13), lambda i: (i, 0)),
            pl.BlockSpec((NUM_SPARSE, BB, EMB_DIM), lambda i: (0, i, 0)),
            _whole((F, F)),
            _whole(bW1.shape), _whole((1, 256)),
            _whole(bW2.shape), _whole((1, 64)),
            _whole(bW3.shape), _whole((1, EMB_DIM)),
            _whole(tW1a.shape), _whole(tW1b.shape), _whole((1, 512)),
            _whole(tW2.shape), _whole((1, 256)),
            _whole(tW3.shape), _whole((1, 1)),
        ],
        out_specs=pl.BlockSpec((BB, 1), lambda i: (i, 0)),
        out_shape=jax.ShapeDtypeStruct((BATCH, 1), jnp.float32),
    )(dense_features, gathered, mask,
      bW1, bb1.reshape(1, 256), bW2, bb2.reshape(1, 64), bW3,
      bb3.reshape(1, EMB_DIM), tW1a, tW1b, tb1.reshape(1, 512), tW2,
      tb2.reshape(1, 256), tW3, tb3.reshape(1, 1))
    return out.reshape(BATCH)


# speedup vs baseline: 1.0038x; 1.0038x over previous
import functools

import jax
import jax.numpy as jnp
from jax import lax
from jax.experimental import pallas as pl
from jax.experimental.pallas import tpu as pltpu
from jax.experimental.pallas import tpu_sc as plsc

NUM_SPARSE = 26
VOCAB_P1 = 100001
EMB_DIM = 16
BATCH = 4096
F = NUM_SPARSE + 1

NW = 32
CHUNK = 128
ROWS = NUM_SPARSE * BATCH
ROWS_PW = ROWS // NW
CHUNKS_PW = ROWS_PW // CHUNK


def _sc_gather(tables_flat, ids_flat):
    mesh = plsc.VectorSubcoreMesh(core_axis_name="c", subcore_axis_name="s")

    @functools.partial(
        pl.kernel,
        out_type=jax.ShapeDtypeStruct((ROWS, EMB_DIM), jnp.float32),
        mesh=mesh,
        scratch_types=[
            pltpu.VMEM((ROWS_PW,), jnp.int32),
            pltpu.VMEM((ROWS_PW, EMB_DIM), jnp.float32),
            pltpu.SemaphoreType.DMA,
        ],
        compiler_params=pltpu.CompilerParams(use_tc_tiling_on_sc=False),
        name="dlrm_sc_gather",
    )
    def gather_kernel(tab_hbm, ids_hbm, out_hbm, idx_v, rows_v, sem):
        wid = lax.axis_index("s") * 2 + lax.axis_index("c")
        base = wid * ROWS_PW
        pltpu.sync_copy(ids_hbm.at[pl.ds(base, ROWS_PW)], idx_v)
        copies = [
            pltpu.async_copy(
                tab_hbm.at[idx_v.at[pl.ds(j * CHUNK, CHUNK)]],
                rows_v.at[pl.ds(j * CHUNK, CHUNK), :],
                sem,
            )
            for j in range(CHUNKS_PW)
        ]
        for c in copies:
            c.wait()
        pltpu.sync_copy(rows_v, out_hbm.at[pl.ds(base, ROWS_PW)])

    return gather_kernel(tables_flat, ids_flat)


BB = 512


def _tc_body(dense_ref, g_ref, mask_ref, bW1_ref, bb1_ref, bW2_ref, bb2_ref,
             bW3_ref, bb3_ref, tW1a_ref, tW1b_ref, tb1_ref, tW2_ref, tb2_ref,
             tW3_ref, tb3_ref, out_ref):
    f32 = jnp.float32
    d = jnp.dot(dense_ref[...], bW1_ref[...], preferred_element_type=f32) + bb1_ref[...]
    d = jnp.dot(d, bW2_ref[...], preferred_element_type=f32) + bb2_ref[...]
    d = jnp.maximum(jnp.dot(d, bW3_ref[...], preferred_element_type=f32) + bb3_ref[...], 0.0)
    feats = jnp.concatenate([g_ref[...], d[None]], axis=0)
    xact = lax.dot_general(feats, feats, (((2,), (2,)), ((1,), (1,))))
    xm = xact * mask_ref[...][None]
    t = jnp.dot(d, tW1a_ref[...], preferred_element_type=f32)
    t = t + jnp.dot(xm.reshape(xm.shape[0], F * F), tW1b_ref[...],
                    preferred_element_type=f32)
    t = t + tb1_ref[...]
    t = jnp.dot(t, tW2_ref[...], preferred_element_type=f32) + tb2_ref[...]
    t = jnp.dot(t, tW3_ref[...], preferred_element_type=f32) + tb3_ref[...]
    out_ref[...] = jax.nn.sigmoid(t)


def _whole(shape):
    return pl.BlockSpec(shape, lambda i: (0,) * len(shape))


def kernel(dense_features, sparse_ids, emb_tables, bW1, bb1, bW2, bb2, bW3,
           bb3, tW1, tb1, tW2, tb2, tW3, tb3):
    tables_flat = emb_tables.reshape(NUM_SPARSE * VOCAB_P1, EMB_DIM)
    offs = (jnp.arange(NUM_SPARSE, dtype=jnp.int32) * VOCAB_P1)[:, None]
    ids_flat = (sparse_ids.astype(jnp.int32) + offs).reshape(ROWS)
    gathered = _sc_gather(tables_flat, ids_flat).reshape(NUM_SPARSE, BATCH, EMB_DIM)

    mask = (lax.broadcasted_iota(jnp.int32, (F, F), 0)
            > lax.broadcasted_iota(jnp.int32, (F, F), 1)).astype(jnp.float32)
    tW1a = tW1[:EMB_DIM]
    tW1b = tW1[EMB_DIM:]

    nb = BATCH // BB
    out = pl.pallas_call(
        _tc_body,
        grid=(nb,),
        in_specs=[
            pl.BlockSpec((BB, 13), lambda i: (i, 0)),
            pl.BlockSpec((NUM_SPARSE, BB, EMB_DIM), lambda i: (0, i, 0)),
            _whole((F, F)),
            _whole(bW1.shape), _whole((1, 256)),
            _whole(bW2.shape), _whole((1, 64)),
            _whole(bW3.shape), _whole((1, EMB_DIM)),
            _whole(tW1a.shape), _whole(tW1b.shape), _whole((1, 512)),
            _whole(tW2.shape), _whole((1, 256)),
            _whole(tW3.shape), _whole((1, 1)),
        ],
        out_specs=pl.BlockSpec((BB, 1), lambda i: (i, 0)),
        out_shape=jax.ShapeDtypeStruct((BATCH, 1), jnp.float32),
    )(dense_features, gathered, mask,
      bW1, bb1.reshape(1, 256), bW2, bb2.reshape(1, 64), bW3,
      bb3.reshape(1, EMB_DIM), tW1a, tW1b, tb1.reshape(1, 512), tW2,
      tb2.reshape(1, 256), tW3, tb3.reshape(1, 1))
    return out.reshape(BATCH)


# speedup vs baseline: 51.2002x; 51.0074x over previous
import functools

import jax
import jax.numpy as jnp
from jax import lax
from jax.experimental import pallas as pl
from jax.experimental.pallas import tpu as pltpu
from jax.experimental.pallas import tpu_sc as plsc

NUM_SPARSE = 26
VOCAB_P1 = 100001
EMB_DIM = 16
BATCH = 4096
F = NUM_SPARSE + 1

NW = 32
NROWS = NUM_SPARSE * EMB_DIM
ROWS_PW = NROWS // NW
GCHUNK = 16


def _sc_gather(tables_t, sparse_ids):
    mesh = plsc.VectorSubcoreMesh(core_axis_name="c", subcore_axis_name="s")

    H0 = 50048
    H1 = VOCAB_P1 - H0

    @functools.partial(
        pl.kernel,
        out_type=jax.ShapeDtypeStruct((NROWS, BATCH), jnp.float32),
        mesh=mesh,
        scratch_types=[
            pltpu.VMEM((H0,), jnp.float32),
            pltpu.VMEM((H1,), jnp.float32),
            pltpu.VMEM((2, BATCH), jnp.int32),
            pltpu.VMEM((2, BATCH), jnp.float32),
            pltpu.SemaphoreType.DMA,
            pltpu.SemaphoreType.DMA,
            pltpu.SemaphoreType.DMA,
            pltpu.SemaphoreType.DMA,
        ],
        compiler_params=pltpu.CompilerParams(use_tc_tiling_on_sc=True,
                                             needs_layout_passes=False),
        name="dlrm_sc_gather",
    )
    def gather_kernel(tab_hbm, ids_hbm, out_hbm, h0_v, h1_v, ids_v, out_v,
                      sem0, sem1, sem_i, sem_o):
        wid = lax.axis_index("s") * 2 + lax.axis_index("c")
        base = wid * ROWS_PW

        def start0(row):
            return pltpu.async_copy(tab_hbm.at[row, pl.ds(0, H0)], h0_v, sem0)

        def start1(row):
            return pltpu.async_copy(
                tab_hbm.at[row, pl.ds(H0, H1)], h1_v, sem1)

        def start_ids(row, buf):
            return pltpu.async_copy(ids_hbm.at[row // EMB_DIM],
                                    ids_v.at[buf], sem_i)

        def make_pass0(cur):
            def pass0(j, carry):
                sl = pl.ds(j * GCHUNK, GCHUNK)
                idx = ids_v[cur, sl]
                m = idx < H0
                g = plsc.load_gather(h0_v, [jnp.minimum(idx, H0 - 1)], mask=m)
                out_v[cur, sl] = g
                return carry
            return pass0

        def make_pass1(cur):
            def pass1(j, carry):
                sl = pl.ds(j * GCHUNK, GCHUNK)
                idx = ids_v[cur, sl]
                m = idx >= H0
                li = jnp.minimum(jnp.maximum(idx - H0, 0), H1 - 1)
                g = plsc.load_gather(h1_v, [li], mask=m)
                out_v[cur, sl] = jnp.where(m, g, out_v[cur, sl])
                return carry
            return pass1

        cp0 = start0(base)
        ids_cp = start_ids(base, 0)
        out_cps = [None, None]
        for k in range(ROWS_PW):
            row = base + k
            cur = k % 2
            ids_cp.wait()
            cp0.wait()
            cp1 = start1(row)
            if out_cps[cur] is not None:
                out_cps[cur].wait()
            lax.fori_loop(0, BATCH // GCHUNK, make_pass0(cur), 0, unroll=8)
            cp1.wait()
            if k + 1 < ROWS_PW:
                cp0 = start0(row + 1)
                ids_cp = start_ids(row + 1, 1 - cur)
            lax.fori_loop(0, BATCH // GCHUNK, make_pass1(cur), 0, unroll=8)
            out_cps[cur] = pltpu.async_copy(out_v.at[cur], out_hbm.at[row],
                                            sem_o)
        out_cps[(ROWS_PW - 1) % 2].wait()
        out_cps[ROWS_PW % 2].wait()

    return gather_kernel(tables_t, sparse_ids)


BB = 1024


def _tc_body(dense_ref, g_ref, bW1_ref, bb1_ref, bW2_ref, bb2_ref,
             bW3_ref, bb3_ref, tW1a_ref, tW1l_ref, tb1_ref, tW2_ref, tb2_ref,
             tW3_ref, tb3_ref, out_ref):
    f32 = jnp.float32
    bf16 = jnp.bfloat16
    d = jnp.dot(dense_ref[...].astype(bf16), bW1_ref[...].astype(bf16),
                preferred_element_type=f32) + bb1_ref[...]
    d = jnp.dot(d.astype(bf16), bW2_ref[...].astype(bf16),
                preferred_element_type=f32) + bb2_ref[...]
    d = jnp.maximum(jnp.dot(d.astype(bf16), bW3_ref[...].astype(bf16),
                            preferred_element_type=f32) + bb3_ref[...], 0.0)
    db = d.astype(bf16)
    feats_t = jnp.concatenate([g_ref[...], jnp.transpose(d)], axis=0)
    slabs = [feats_t[EMB_DIM * i:EMB_DIM * (i + 1)] for i in range(F)]
    rows = []
    for f in range(1, F):
        for g in range(f):
            rows.append(jnp.sum(slabs[f] * slabs[g], axis=0))
    act_t = jnp.stack(rows, axis=0)
    t = jnp.dot(db, tW1a_ref[...].astype(bf16), preferred_element_type=f32)
    t = t + lax.dot_general(act_t.astype(bf16), tW1l_ref[...].astype(bf16),
                            (((0,), (0,)), ((), ())),
                            preferred_element_type=f32)
    t = t + tb1_ref[...]
    t = jnp.dot(t.astype(bf16), tW2_ref[...].astype(bf16),
                preferred_element_type=f32) + tb2_ref[...]
    t = jnp.dot(t.astype(bf16), tW3_ref[...].astype(bf16),
                preferred_element_type=f32) + tb3_ref[...]
    out_ref[...] = jax.nn.sigmoid(t)


def _whole(shape):
    return pl.BlockSpec(shape, lambda i: (0,) * len(shape))


def kernel(dense_features, sparse_ids, emb_tables, bW1, bb1, bW2, bb2, bW3,
           bb3, tW1, tb1, tW2, tb2, tW3, tb3):
    tables_t = jnp.transpose(emb_tables, (0, 2, 1)).reshape(NROWS, VOCAB_P1)
    gathered = _sc_gather(tables_t, sparse_ids)

    tW1a = tW1[:EMB_DIM]
    pair_rows = jnp.asarray([F * f + g for f in range(1, F) for g in range(f)],
                            dtype=jnp.int32)
    tW1l = tW1[EMB_DIM:][pair_rows]

    nb = BATCH // BB
    out = pl.pallas_call(
        _tc_body,
        grid=(nb,),
        in_specs=[
            pl.BlockSpec((BB, 13), lambda i: (i, 0)),
            pl.BlockSpec((NROWS, BB), lambda i: (0, i)),
            _whole(bW1.shape), _whole((1, 256)),
            _whole(bW2.shape), _whole((1, 64)),
            _whole(bW3.shape), _whole((1, EMB_DIM)),
            _whole(tW1a.shape), _whole(tW1l.shape), _whole((1, 512)),
            _whole(tW2.shape), _whole((1, 256)),
            _whole(tW3.shape), _whole((1, 1)),
        ],
        out_specs=pl.BlockSpec((BB, 1), lambda i: (i, 0)),
        out_shape=jax.ShapeDtypeStruct((BATCH, 1), jnp.float32),
    )(dense_features, gathered,
      bW1, bb1.reshape(1, 256), bW2, bb2.reshape(1, 64),
      bW3, bb3.reshape(1, EMB_DIM), tW1a, tW1l,
      tb1.reshape(1, 512), tW2, tb2.reshape(1, 256), tW3, tb3.reshape(1, 1))
    return out.reshape(BATCH)
